# baseline (speedup 1.0000x reference)
import functools

import jax
import jax.numpy as jnp
from jax import lax
from jax.experimental import pallas as pl
from jax.experimental.pallas import tpu as pltpu
from jax.experimental.pallas import tpu_sc as plsc

N = 10000
H = 512
E_EXTRA = 1024
NCH = 96
CH = 112
OUT_ROWS = 120
DUMMY = 116
K = 64
M_BLK = 1000
LAST_CID = (N - 1) // CH
LAST_ROWS = N - LAST_CID * CH


def _agg_body(F, h_hbm, epk_hbm, off_hbm, zeros_hbm, out_hbm,
              offv, idx0, idx1, srcm0, srcm1, dstm0, dstm1, rows0, rows1,
              outbuf, sg0, sg1, si0, si1):
    c = lax.axis_index("c")
    s = lax.axis_index("s")
    w = 2 * s + c
    iota = lax.iota(jnp.int32, 16)
    pltpu.sync_copy(off_hbm, offv)

    idxb = (idx0, idx1)
    srcmb = (srcm0, srcm1)
    dstmb = (dstm0, dstm1)
    rowsb = (rows0, rows1)
    sgb = (sg0, sg1)
    sib = (si0, si1)

    def chunk_body(k, carry):
        cid = 3 * w + k
        ov = offv[pl.ds(cid, 16)]
        o0 = ov[0]
        o1 = ov[1]
        astart = o0 & (-64)
        row0 = lax.div(astart, 64)
        nb = lax.div(o1 - astart + (K - 1), K)

        pltpu.sync_copy(zeros_hbm, outbuf.at[pl.ds(0, CH)])

        def stage_idx(i, buf, sem):
            b = pl.multiple_of((row0 + i) * 128, 128)
            return pltpu.async_copy(epk_hbm.at[pl.ds(b, 128)], buf, sem)

        def mask_batch(i, p):
            ebase = astart + i * K
            for j in range(K // 16):
                e = ebase + j * 16 + iota
                valid = (e >= o0) & (e < o1)
                sv = idxb[p][pl.ds(j * 16, 16)]
                dv = idxb[p][pl.ds(64 + j * 16, 16)]
                srcmb[p][pl.ds(j * 16, 16)] = jnp.where(valid, sv, 0)
                dstmb[p][pl.ds(j * 16, 16)] = jnp.where(valid, dv, DUMMY)

        @pl.when(nb > 0)
        def _():
            stage_idx(0, idxb[0], sib[0]).wait()
            mask_batch(0, 0)
            pltpu.async_copy(h_hbm.at[srcmb[0]], rowsb[0], sgb[0])

        @pl.when(nb > 1)
        def _():
            stage_idx(1, idxb[1], sib[1])

        def pair_body(ih, carry2):
            for bpar in range(2):
                i = ih * 2 + bpar
                cur, nxt = bpar, 1 - bpar

                @pl.when(i + 1 < nb)
                def _():
                    pltpu.make_async_copy(
                        epk_hbm.at[pl.ds(0, 128)], idxb[nxt], sib[nxt]).wait()
                    mask_batch(i + 1, nxt)
                    pltpu.async_copy(h_hbm.at[srcmb[nxt]], rowsb[nxt],
                                     sgb[nxt])

                @pl.when(i + 2 < nb)
                def _():
                    stage_idx(i + 2, idxb[cur], sib[cur])

                @pl.when(i < nb)
                def _():
                    pltpu.make_async_copy(
                        h_hbm.at[srcmb[cur]], rowsb[cur], sgb[cur]).wait()

                    def sub(j, c3):
                        dm = dstmb[cur][pl.ds(j * 16, 16)]
                        for l in range(16):
                            d = dm[l]
                            r = j * 16 + l
                            for m in range(F // 16):
                                outbuf[d, pl.ds(m * 16, 16)] = (
                                    outbuf[d, pl.ds(m * 16, 16)]
                                    + rowsb[cur][r, pl.ds(m * 16, 16)])
                        return c3

                    lax.fori_loop(0, K // 16, sub, 0)
            return carry2

        lax.fori_loop(0, lax.div(nb + 1, 2), pair_body, 0)

        @pl.when(cid < LAST_CID)
        def _():
            pltpu.sync_copy(
                outbuf.at[pl.ds(0, CH)],
                out_hbm.at[pl.ds(pl.multiple_of(cid * CH, 8), CH)])

        @pl.when(cid == LAST_CID)
        def _():
            pltpu.sync_copy(outbuf.at[pl.ds(0, LAST_ROWS)],
                            out_hbm.at[pl.ds(LAST_CID * CH, LAST_ROWS)])

        return carry

    lax.fori_loop(0, 3, chunk_body, 0)


@functools.cache
def _make_agg(F):
    mesh = plsc.VectorSubcoreMesh(core_axis_name="c", subcore_axis_name="s")
    return pl.kernel(
        functools.partial(_agg_body, F),
        out_type=jax.ShapeDtypeStruct((N, F), jnp.float32),
        mesh=mesh,
        scratch_types=[
            pltpu.VMEM((128,), jnp.int32),
            pltpu.VMEM((128,), jnp.int32),
            pltpu.VMEM((128,), jnp.int32),
            pltpu.VMEM((K,), jnp.int32),
            pltpu.VMEM((K,), jnp.int32),
            pltpu.VMEM((K,), jnp.int32),
            pltpu.VMEM((K,), jnp.int32),
            pltpu.VMEM((K, F), jnp.float32),
            pltpu.VMEM((K, F), jnp.float32),
            pltpu.VMEM((OUT_ROWS, F), jnp.float32),
            pltpu.SemaphoreType.DMA,
            pltpu.SemaphoreType.DMA,
            pltpu.SemaphoreType.DMA,
            pltpu.SemaphoreType.DMA,
        ],
    )


def _prep(edge_index):
    src, dst = edge_index[0], edge_index[1]
    e = src.shape[0]
    order = jnp.argsort(dst)
    src_s = jnp.take(src, order).astype(jnp.int32)
    dst_s = jnp.take(dst, order)
    dloc = (dst_s - (dst_s // CH) * CH).astype(jnp.int32)
    pad = jnp.zeros((E_EXTRA,), dtype=jnp.int32)
    src_p = jnp.concatenate([src_s, pad]).reshape(-1, K)
    dloc_p = jnp.concatenate([dloc, pad]).reshape(-1, K)
    epk = jnp.concatenate([src_p, dloc_p], axis=1).reshape(-1)
    bounds = jnp.searchsorted(dst_s, jnp.arange(NCH + 1, dtype=jnp.int32) * CH)
    off = jnp.concatenate(
        [bounds.astype(jnp.int32), jnp.full((128 - NCH - 1,), e, jnp.int32)])
    rowptr = jnp.searchsorted(dst_s, jnp.arange(N + 1, dtype=jnp.int32))
    cnt = (rowptr[1:] - rowptr[:-1]).astype(jnp.float32)
    inv = (1.0 / jnp.maximum(cnt, 1.0)).reshape(N, 1)
    return epk, off, inv


def _sage_mm_body(agg_ref, inv_ref, h_ref, wl_ref, wr_ref, b_ref, out_ref):
    mean = agg_ref[...] * inv_ref[...]
    yl = lax.dot_general(mean, wl_ref[...], (((1,), (1,)), ((), ())),
                         preferred_element_type=jnp.float32)
    yr = lax.dot_general(h_ref[...], wr_ref[...], (((1,), (1,)), ((), ())),
                         preferred_element_type=jnp.float32)
    out_ref[...] = jnp.maximum(yl + yr + b_ref[...], 0.0)


def _sage_mm(agg, inv, h, wl, wr, b):
    f_in = h.shape[1]
    grid = N // M_BLK
    return pl.pallas_call(
        _sage_mm_body,
        grid=(grid,),
        in_specs=[
            pl.BlockSpec((M_BLK, f_in), lambda i: (i, 0)),
            pl.BlockSpec((M_BLK, 1), lambda i: (i, 0)),
            pl.BlockSpec((M_BLK, f_in), lambda i: (i, 0)),
            pl.BlockSpec((H, f_in), lambda i: (0, 0)),
            pl.BlockSpec((H, f_in), lambda i: (0, 0)),
            pl.BlockSpec((1, H), lambda i: (0, 0)),
        ],
        out_specs=pl.BlockSpec((M_BLK, H), lambda i: (i, 0)),
        out_shape=jax.ShapeDtypeStruct((N, H), jnp.float32),
    )(agg, inv, h, wl, wr, b.reshape(1, H))


def _layer(h, epk, off, zeros, inv, wl, wr, b):
    agg = _make_agg(h.shape[1])(h, epk, off, zeros)
    return _sage_mm(agg, inv, h, wl, wr, b)


def kernel(x, edge_index_connections, edge_index_destinations,
           W1l, W1r, b1, W2l, W2r, b2, W3l, W3r, b3, W4l, W4r, b4):
    epk_c, off_c, inv_c = _prep(edge_index_connections)
    epk_d, off_d, inv_d = _prep(edge_index_destinations)

    z256 = jnp.zeros((CH, 256), jnp.float32)
    z512 = jnp.zeros((CH, 512), jnp.float32)

    h = _layer(x, epk_c, off_c, z256, inv_c, W1l, W1r, b1)
    h = _layer(h, epk_c, off_c, z512, inv_c, W4l, W4r, b4)
    h = _layer(h, epk_c, off_c, z512, inv_c, W4l, W4r, b4)
    h = _layer(h, epk_d, off_d, z512, inv_d, W2l, W2r, b2)
    h = _layer(h, epk_c, off_c, z512, inv_c, W3l, W3r, b3)
    h = _layer(h, epk_c, off_c, z512, inv_c, W3l, W3r, b3)
    return h


# speedup vs baseline: 1.5026x; 1.5026x over previous
import functools

import jax
import jax.numpy as jnp
from jax import lax
from jax.experimental import pallas as pl
from jax.experimental.pallas import tpu as pltpu
from jax.experimental.pallas import tpu_sc as plsc

N = 10000
H = 512
E_EXTRA = 1024
NCH = 128
CH = 80
OUT_ROWS = 88
DUMMY = 84
K = 64
M_BLK = 1000
LAST_CID = N // CH - 1


def _agg_body(F, h_hbm, epk_hbm, off_hbm, zeros_hbm, out_hbm,
              offv, idx0, idx1, srcm0, srcm1, dstm0, dstm1, rows0, rows1,
              outbuf, accbuf, prevbuf, sg0, sg1, si0, si1):
    c = lax.axis_index("c")
    s = lax.axis_index("s")
    w = 2 * s + c
    iota = lax.iota(jnp.int32, 16)
    nm = F // 16
    pltpu.sync_copy(off_hbm, offv)

    idxb = (idx0, idx1)
    srcmb = (srcm0, srcm1)
    dstmb = (dstm0, dstm1)
    rowsb = (rows0, rows1)
    sgb = (sg0, sg1)
    sib = (si0, si1)

    def chunk_body(k, carry):
        cid = 4 * w + k
        ov = offv[pl.ds(cid, 16)]
        o0 = ov[0]
        o1 = ov[1]
        astart = o0 & (-64)
        row0 = lax.div(astart, 64)
        nb = lax.div(o1 - astart + (K - 1), K)

        pltpu.sync_copy(zeros_hbm, outbuf.at[pl.ds(0, CH)])
        prevbuf[pl.ds(0, 16)] = jnp.full((16,), DUMMY, jnp.int32)
        for m in range(nm):
            accbuf[m, pl.ds(0, 16)] = jnp.zeros((16,), jnp.float32)

        def stage_idx(i, buf, sem):
            b = pl.multiple_of((row0 + i) * 128, 128)
            return pltpu.async_copy(epk_hbm.at[pl.ds(b, 128)], buf, sem)

        def mask_batch(i, p):
            ebase = astart + i * K
            for j in range(K // 16):
                e = ebase + j * 16 + iota
                valid = (e >= o0) & (e < o1)
                sv = idxb[p][pl.ds(j * 16, 16)]
                dv = idxb[p][pl.ds(64 + j * 16, 16)]
                srcmb[p][pl.ds(j * 16, 16)] = jnp.where(valid, sv, 0)
                dstmb[p][pl.ds(j * 16, 16)] = jnp.where(valid, dv, DUMMY)

        @pl.when(nb > 0)
        def _():
            stage_idx(0, idxb[0], sib[0]).wait()
            mask_batch(0, 0)
            pltpu.async_copy(h_hbm.at[srcmb[0]], rowsb[0], sgb[0])

        @pl.when(nb > 1)
        def _():
            stage_idx(1, idxb[1], sib[1])

        def pair_body(ih, carry2):
            for bpar in range(2):
                i = ih * 2 + bpar
                cur, nxt = bpar, 1 - bpar

                @pl.when(i + 1 < nb)
                def _():
                    pltpu.make_async_copy(
                        epk_hbm.at[pl.ds(0, 128)], idxb[nxt], sib[nxt]).wait()
                    mask_batch(i + 1, nxt)
                    pltpu.async_copy(h_hbm.at[srcmb[nxt]], rowsb[nxt],
                                     sgb[nxt])

                @pl.when(i + 2 < nb)
                def _():
                    stage_idx(i + 2, idxb[cur], sib[cur])

                @pl.when(i < nb)
                def _():
                    pltpu.make_async_copy(
                        h_hbm.at[srcmb[cur]], rowsb[cur], sgb[cur]).wait()

                    pv = prevbuf[pl.ds(0, 16)]
                    state = [pv[0]]
                    for m in range(nm):
                        state.append(accbuf[m, pl.ds(0, 16)])

                    def sub(j, st):
                        prev_d = st[0]
                        acc = list(st[1:])
                        dm = dstmb[cur][pl.ds(j * 16, 16)]
                        rbase = j * 16
                        for l in range(16):
                            d = dm[l]
                            re = [rowsb[cur][rbase + l, pl.ds(m * 16, 16)]
                                  for m in range(nm)]
                            is_new = d != prev_d

                            @pl.when(is_new)
                            def _():
                                for m in range(nm):
                                    outbuf[prev_d, pl.ds(m * 16, 16)] = acc[m]

                            acc = [jnp.where(is_new, r_, a_ + r_)
                                   for a_, r_ in zip(acc, re)]
                            prev_d = d
                        return (prev_d, *acc)

                    st = lax.fori_loop(0, K // 16, sub, tuple(state))
                    prevbuf[pl.ds(0, 16)] = jnp.full((16,), 0, jnp.int32) + st[0]
                    for m in range(nm):
                        accbuf[m, pl.ds(0, 16)] = st[1 + m]
            return carry2

        lax.fori_loop(0, lax.div(nb + 1, 2), pair_body, 0)

        pv = prevbuf[pl.ds(0, 16)]
        last_d = pv[0]
        for m in range(nm):
            outbuf[last_d, pl.ds(m * 16, 16)] = accbuf[m, pl.ds(0, 16)]

        @pl.when(cid <= LAST_CID)
        def _():
            pltpu.sync_copy(
                outbuf.at[pl.ds(0, CH)],
                out_hbm.at[pl.ds(pl.multiple_of(cid * CH, 8), CH)])

        return carry

    lax.fori_loop(0, 4, chunk_body, 0)


@functools.cache
def _make_agg(F):
    mesh = plsc.VectorSubcoreMesh(core_axis_name="c", subcore_axis_name="s")
    return pl.kernel(
        functools.partial(_agg_body, F),
        out_type=jax.ShapeDtypeStruct((N, F), jnp.float32),
        mesh=mesh,
        scratch_types=[
            pltpu.VMEM((160,), jnp.int32),
            pltpu.VMEM((128,), jnp.int32),
            pltpu.VMEM((128,), jnp.int32),
            pltpu.VMEM((K,), jnp.int32),
            pltpu.VMEM((K,), jnp.int32),
            pltpu.VMEM((K,), jnp.int32),
            pltpu.VMEM((K,), jnp.int32),
            pltpu.VMEM((K, F), jnp.float32),
            pltpu.VMEM((K, F), jnp.float32),
            pltpu.VMEM((OUT_ROWS, F), jnp.float32),
            pltpu.VMEM((32, 16), jnp.float32),
            pltpu.VMEM((16,), jnp.int32),
            pltpu.SemaphoreType.DMA,
            pltpu.SemaphoreType.DMA,
            pltpu.SemaphoreType.DMA,
            pltpu.SemaphoreType.DMA,
        ],
    )


def _prep(edge_index):
    src, dst = edge_index[0], edge_index[1]
    e = src.shape[0]
    order = jnp.argsort(dst)
    src_s = jnp.take(src, order).astype(jnp.int32)
    dst_s = jnp.take(dst, order)
    dloc = (dst_s - (dst_s // CH) * CH).astype(jnp.int32)
    pad = jnp.zeros((E_EXTRA,), dtype=jnp.int32)
    src_p = jnp.concatenate([src_s, pad]).reshape(-1, K)
    dloc_p = jnp.concatenate([dloc, pad]).reshape(-1, K)
    epk = jnp.concatenate([src_p, dloc_p], axis=1).reshape(-1)
    bounds = jnp.searchsorted(dst_s, jnp.arange(NCH + 1, dtype=jnp.int32) * CH)
    off = jnp.concatenate(
        [bounds.astype(jnp.int32), jnp.full((160 - NCH - 1,), e, jnp.int32)])
    rowptr = jnp.searchsorted(dst_s, jnp.arange(N + 1, dtype=jnp.int32))
    cnt = (rowptr[1:] - rowptr[:-1]).astype(jnp.float32)
    inv = (1.0 / jnp.maximum(cnt, 1.0)).reshape(N, 1)
    return epk, off, inv


def _sage_mm_body(agg_ref, inv_ref, h_ref, wl_ref, wr_ref, b_ref, out_ref):
    mean = agg_ref[...] * inv_ref[...]
    yl = lax.dot_general(mean, wl_ref[...], (((1,), (1,)), ((), ())),
                         preferred_element_type=jnp.float32)
    yr = lax.dot_general(h_ref[...], wr_ref[...], (((1,), (1,)), ((), ())),
                         preferred_element_type=jnp.float32)
    out_ref[...] = jnp.maximum(yl + yr + b_ref[...], 0.0)


def _sage_mm(agg, inv, h, wl, wr, b):
    f_in = h.shape[1]
    grid = N // M_BLK
    return pl.pallas_call(
        _sage_mm_body,
        grid=(grid,),
        in_specs=[
            pl.BlockSpec((M_BLK, f_in), lambda i: (i, 0)),
            pl.BlockSpec((M_BLK, 1), lambda i: (i, 0)),
            pl.BlockSpec((M_BLK, f_in), lambda i: (i, 0)),
            pl.BlockSpec((H, f_in), lambda i: (0, 0)),
            pl.BlockSpec((H, f_in), lambda i: (0, 0)),
            pl.BlockSpec((1, H), lambda i: (0, 0)),
        ],
        out_specs=pl.BlockSpec((M_BLK, H), lambda i: (i, 0)),
        out_shape=jax.ShapeDtypeStruct((N, H), jnp.float32),
    )(agg, inv, h, wl, wr, b.reshape(1, H))


def _layer(h, epk, off, zeros, inv, wl, wr, b):
    agg = _make_agg(h.shape[1])(h, epk, off, zeros)
    return _sage_mm(agg, inv, h, wl, wr, b)


def kernel(x, edge_index_connections, edge_index_destinations,
           W1l, W1r, b1, W2l, W2r, b2, W3l, W3r, b3, W4l, W4r, b4):
    epk_c, off_c, inv_c = _prep(edge_index_connections)
    epk_d, off_d, inv_d = _prep(edge_index_destinations)

    z256 = jnp.zeros((CH, 256), jnp.float32)
    z512 = jnp.zeros((CH, 512), jnp.float32)

    h = _layer(x, epk_c, off_c, z256, inv_c, W1l, W1r, b1)
    h = _layer(h, epk_c, off_c, z512, inv_c, W4l, W4r, b4)
    h = _layer(h, epk_c, off_c, z512, inv_c, W4l, W4r, b4)
    h = _layer(h, epk_d, off_d, z512, inv_d, W2l, W2r, b2)
    h = _layer(h, epk_c, off_c, z512, inv_c, W3l, W3r, b3)
    h = _layer(h, epk_c, off_c, z512, inv_c, W3l, W3r, b3)
    return h
